# baseline (speedup 1.0000x reference)
import functools

import jax
import jax.numpy as jnp
import numpy as np
from jax import lax
from jax.experimental import pallas as pl
from jax.experimental.pallas import tpu as pltpu
from jax.experimental.pallas import tpu_sc as plsc

NN = 10000
NE = 320000
D = 128
NC = 2
NS = 16
NW = NC * NS
E_PER_W = NE // NW
CHUNK = 80
NCHUNK = E_PER_W // CHUNK
OUT_TILES = 10
ROWS_PER_OTILE = NN // OUT_TILES
ZROWS = 40

_QPERM = np.empty((D,), dtype=np.int32)
for _j in range(D // 32):
    for _k in range(16):
        _QPERM[32 * _j + _k] = 32 * _j + 2 * _k
        _QPERM[32 * _j + 16 + _k] = 32 * _j + 2 * _k + 1


def _bcast_lane(v, k):
    return lax.gather(
        v, jnp.full((16, 1), k, jnp.int32),
        lax.GatherDimensionNumbers(
            offset_dims=(), collapsed_slice_dims=(0,), start_index_map=(0,)),
        slice_sizes=(1,),
        mode=lax.GatherScatterMode.PROMISE_IN_BOUNDS)


def _sc_body(h_hbm, src_hbm, dst_hbm, w_hbm, out_hbm,
             src_l, dst_l, w_l, rin0, rin1, rout, agg, sem0, sem1, sems):
    c = lax.axis_index("c")
    s = lax.axis_index("s")
    wid = c * NS + s

    cp_src = pltpu.async_copy(src_hbm.at[wid, 0], src_l, sems)
    cp_dst = pltpu.async_copy(dst_hbm.at[wid], dst_l, sems)
    cp_w = pltpu.async_copy(w_hbm.at[wid, 0], w_l, sems)

    def zfill(i, carry):
        for j in range(D // 16):
            rout[i, pl.ds(j * 16, 16)] = jnp.zeros((16,), jnp.float32)
        return carry
    lax.fori_loop(0, ZROWS, zfill, 0)

    cp_src.wait()
    cp_dst.wait()
    cp_w.wait()

    pltpu.async_copy(h_hbm.at[src_l.at[pl.ds(0, CHUNK)]], rin0, sem0)

    @pl.when(s < OUT_TILES)
    def _zero():
        zcopies = []
        for k in range(ROWS_PER_OTILE // ZROWS):
            zcopies.append(pltpu.async_copy(
                rout.at[pl.ds(0, ZROWS)],
                agg.at[pl.ds(s * ROWS_PER_OTILE + k * ZROWS, ZROWS)], sems))
        for z in zcopies:
            z.wait()
    plsc.subcore_barrier()

    bufs = (rin0, rin1)
    dsems = (sem0, sem1)
    himask = jnp.full((16,), -65536, jnp.int32)

    def chunk_body(i, carry):
        for p in range(2):
            @pl.when((i % 2) == p)
            def _do(p=p):
                cur, nxt = bufs[p], bufs[1 - p]

                @pl.when(i + 1 < NCHUNK)
                def _prefetch():
                    pltpu.async_copy(
                        h_hbm.at[src_l.at[pl.ds((i + 1) * CHUNK, CHUNK)]],
                        nxt, dsems[1 - p])

                pltpu.make_async_copy(
                    h_hbm.at[src_l.at[pl.ds(0, CHUNK)]], cur, dsems[p]).wait()

                def group(g, gcarry):
                    wv16 = w_l[pl.ds(i * CHUNK + g * 16, 16)]
                    for k in range(16):
                        wv = _bcast_lane(wv16, k)
                        e = g * 16 + k
                        for j in range(D // 32):
                            xi = cur[e, pl.ds(j * 16, 16)]
                            lo = plsc.bitcast(
                                lax.shift_left(xi, 16), jnp.float32)
                            hi = plsc.bitcast(
                                lax.bitwise_and(xi, himask), jnp.float32)
                            rout[e, pl.ds(j * 32, 16)] = lo * wv
                            rout[e, pl.ds(j * 32 + 16, 16)] = hi * wv
                    return gcarry
                lax.fori_loop(0, CHUNK // 16, group, 0)

                pltpu.sync_copy(rout, agg.at[dst_l.at[i]], add=True)
        return carry
    lax.fori_loop(0, NCHUNK, chunk_body, 0)

    plsc.subcore_barrier()

    @pl.when(s < OUT_TILES)
    def _copy_out():
        pltpu.sync_copy(
            agg.at[pl.ds(s * ROWS_PER_OTILE, ROWS_PER_OTILE)],
            out_hbm.at[pl.ds(c * NN + s * ROWS_PER_OTILE, ROWS_PER_OTILE)])


_sc_aggregate = functools.partial(
    pl.kernel,
    _sc_body,
    out_type=jax.ShapeDtypeStruct((NC * NN, D), jnp.float32),
    mesh=plsc.VectorSubcoreMesh(core_axis_name="c", subcore_axis_name="s"),
    scratch_types=[
        pltpu.VMEM((E_PER_W,), jnp.int32),
        pltpu.VMEM((NCHUNK, CHUNK), jnp.int32),
        pltpu.VMEM((E_PER_W,), jnp.float32),
        pltpu.VMEM((CHUNK, D // 2), jnp.int32),
        pltpu.VMEM((CHUNK, D // 2), jnp.int32),
        pltpu.VMEM((CHUNK, D), jnp.float32),
        pltpu.VMEM_SHARED((NN, D), jnp.float32),
        pltpu.SemaphoreType.DMA,
        pltpu.SemaphoreType.DMA,
        pltpu.SemaphoreType.DMA,
    ],
    compiler_params=pltpu.CompilerParams(
        use_tc_tiling_on_sc=False, needs_layout_passes=False),
)()


def _mm_body(p0_ref, p1_ref, w_ref, b_ref, o_ref, obf_ref, *, act):
    x = p0_ref[...] + p1_ref[...]
    y = jnp.dot(x, w_ref[...], preferred_element_type=jnp.float32) + b_ref[...]
    y = jnp.tanh(y) if act else y
    o_ref[...] = y
    obf_ref[...] = y.astype(jnp.bfloat16)


def _tc_layer(p0, p1, Wq, b, act):
    R = 2000
    return pl.pallas_call(
        functools.partial(_mm_body, act=act),
        grid=(NN // R,),
        in_specs=[
            pl.BlockSpec((R, D), lambda i: (i, 0)),
            pl.BlockSpec((R, D), lambda i: (i, 0)),
            pl.BlockSpec((D, D), lambda i: (0, 0)),
            pl.BlockSpec((1, D), lambda i: (0, 0)),
        ],
        out_specs=[
            pl.BlockSpec((R, D), lambda i: (i, 0)),
            pl.BlockSpec((R, D), lambda i: (i, 0)),
        ],
        out_shape=[
            jax.ShapeDtypeStruct((NN, D), jnp.float32),
            jax.ShapeDtypeStruct((NN, D), jnp.bfloat16),
        ],
    )(p0, p1, Wq, b.reshape(1, D))


def kernel(h, edge_index, edge_weight, W0, b0, W1, b1, W2, b2):
    src3 = edge_index[0].astype(jnp.int32).reshape(NW, 1, E_PER_W)
    dst3 = edge_index[1].astype(jnp.int32).reshape(NW, NCHUNK, CHUNK)
    w3 = edge_weight.astype(jnp.float32).reshape(NW, 1, E_PER_W)
    qperm = jnp.asarray(_QPERM)

    def as_i32(t_bf16):
        return lax.bitcast_convert_type(
            t_bf16.reshape(NN, D // 2, 2), jnp.int32)

    layers = [(W0, b0), (W1, b1), (W2, b2)]
    outs = [h]
    table = as_i32(h.astype(jnp.bfloat16))
    for l, (W, b) in enumerate(layers):
        part = _sc_aggregate(table, src3, dst3, w3)
        cur, tbf = _tc_layer(part[:NN], part[NN:], W[qperm, :], b,
                             act=(l < 2))
        table = as_i32(tbf)
        outs.append(cur)
    return jnp.concatenate(outs, axis=1)


# speedup vs baseline: 2.2438x; 2.2438x over previous
import functools

import jax
import jax.numpy as jnp
from jax import lax
from jax.experimental import pallas as pl
from jax.experimental.pallas import tpu as pltpu
from jax.experimental.pallas import tpu_sc as plsc

NN = 10000
NE = 320000
D = 128
NC = 2
NS = 16
NW = NC * NS
E_PER_W = NE // NW
CHUNK = 80
NCHUNK = E_PER_W // CHUNK
OUT_TILES = 10
ROWS_PER_OTILE = NN // OUT_TILES
ZROWS = 40


def _bcast_lane(v, k):
    return lax.gather(
        v, jnp.full((16, 1), k, jnp.int32),
        lax.GatherDimensionNumbers(
            offset_dims=(), collapsed_slice_dims=(0,), start_index_map=(0,)),
        slice_sizes=(1,),
        mode=lax.GatherScatterMode.PROMISE_IN_BOUNDS)


def _sc_body(h_hbm, src_hbm, dst_hbm, w_hbm, out_hbm,
             src_l, dst_l, w_l, rows0, rows1, agg,
             sem0, sem1, ssem0, ssem1, sems):
    c = lax.axis_index("c")
    s = lax.axis_index("s")
    wid = c * NS + s

    cp_src = pltpu.async_copy(src_hbm.at[wid, 0], src_l, sems)
    cp_dst = pltpu.async_copy(dst_hbm.at[wid], dst_l, sems)
    cp_w = pltpu.async_copy(w_hbm.at[wid, 0], w_l, sems)

    def zfill(i, carry):
        for j in range(D // 16):
            rows1[i, pl.ds(j * 16, 16)] = jnp.zeros((16,), jnp.float32)
        return carry
    lax.fori_loop(0, ZROWS, zfill, 0)

    cp_src.wait()
    cp_dst.wait()
    cp_w.wait()

    pltpu.async_copy(h_hbm.at[src_l.at[pl.ds(0, CHUNK)]], rows0, sem0)

    @pl.when(s < OUT_TILES)
    def _zero():
        zcopies = []
        for k in range(ROWS_PER_OTILE // ZROWS):
            zcopies.append(pltpu.async_copy(
                rows1.at[pl.ds(0, ZROWS)],
                agg.at[pl.ds(s * ROWS_PER_OTILE + k * ZROWS, ZROWS)], sems))
        for z in zcopies:
            z.wait()
    plsc.subcore_barrier()

    bufs = (rows0, rows1)
    dsems = (sem0, sem1)
    ssems = (ssem0, ssem1)

    def chunk_body(i, carry):
        for p in range(2):
            @pl.when((i % 2) == p)
            def _do(p=p):
                cur, nxt = bufs[p], bufs[1 - p]

                @pl.when(i > 0)
                def _retire():
                    pltpu.make_async_copy(
                        nxt, agg.at[dst_l.at[i]], ssems[1 - p]).wait()

                @pl.when(i + 1 < NCHUNK)
                def _prefetch():
                    pltpu.async_copy(
                        h_hbm.at[src_l.at[pl.ds((i + 1) * CHUNK, CHUNK)]],
                        nxt, dsems[1 - p])

                pltpu.make_async_copy(
                    h_hbm.at[src_l.at[pl.ds(0, CHUNK)]], cur, dsems[p]).wait()

                def group(g, gcarry):
                    wv16 = w_l[pl.ds(i * CHUNK + g * 16, 16)]
                    for k in range(16):
                        wv = _bcast_lane(wv16, k)
                        e = g * 16 + k
                        for j in range(D // 16):
                            sl = pl.ds(j * 16, 16)
                            cur[e, sl] = cur[e, sl] * wv
                    return gcarry
                lax.fori_loop(0, CHUNK // 16, group, 0)

                pltpu.async_copy(cur, agg.at[dst_l.at[i]], ssems[p], add=True)
        return carry
    lax.fori_loop(0, NCHUNK, chunk_body, 0)

    pltpu.make_async_copy(
        bufs[(NCHUNK - 1) % 2], agg.at[dst_l.at[NCHUNK - 1]],
        ssems[(NCHUNK - 1) % 2]).wait()

    plsc.subcore_barrier()

    @pl.when(s < OUT_TILES)
    def _copy_out():
        pltpu.sync_copy(
            agg.at[pl.ds(s * ROWS_PER_OTILE, ROWS_PER_OTILE)],
            out_hbm.at[pl.ds(c * NN + s * ROWS_PER_OTILE, ROWS_PER_OTILE)])


_sc_aggregate = functools.partial(
    pl.kernel,
    _sc_body,
    out_type=jax.ShapeDtypeStruct((NC * NN, D), jnp.float32),
    mesh=plsc.VectorSubcoreMesh(core_axis_name="c", subcore_axis_name="s"),
    scratch_types=[
        pltpu.VMEM((E_PER_W,), jnp.int32),
        pltpu.VMEM((NCHUNK, CHUNK), jnp.int32),
        pltpu.VMEM((E_PER_W,), jnp.float32),
        pltpu.VMEM((CHUNK, D), jnp.float32),
        pltpu.VMEM((CHUNK, D), jnp.float32),
        pltpu.VMEM_SHARED((NN, D), jnp.float32),
        pltpu.SemaphoreType.DMA,
        pltpu.SemaphoreType.DMA,
        pltpu.SemaphoreType.DMA,
        pltpu.SemaphoreType.DMA,
        pltpu.SemaphoreType.DMA,
    ],
    compiler_params=pltpu.CompilerParams(use_tc_tiling_on_sc=False),
)()


def _mm_body(p0_ref, p1_ref, w_ref, b_ref, o_ref, *, act):
    x = p0_ref[...] + p1_ref[...]
    y = jnp.dot(x, w_ref[...], preferred_element_type=jnp.float32) + b_ref[...]
    o_ref[...] = jnp.tanh(y) if act else y


def _tc_layer(part, W, b, act):
    R = 2000
    G = NN // R
    return pl.pallas_call(
        functools.partial(_mm_body, act=act),
        grid=(G,),
        in_specs=[
            pl.BlockSpec((R, D), lambda i: (i, 0)),
            pl.BlockSpec((R, D), lambda i: (i + G, 0)),
            pl.BlockSpec((D, D), lambda i: (0, 0)),
            pl.BlockSpec((1, D), lambda i: (0, 0)),
        ],
        out_specs=pl.BlockSpec((R, D), lambda i: (i, 0)),
        out_shape=jax.ShapeDtypeStruct((NN, D), jnp.float32),
    )(part, part, W, b.reshape(1, D))


def kernel(h, edge_index, edge_weight, W0, b0, W1, b1, W2, b2):
    src3 = edge_index[0].astype(jnp.int32).reshape(NW, 1, E_PER_W)
    dst3 = edge_index[1].astype(jnp.int32).reshape(NW, NCHUNK, CHUNK)
    w3 = edge_weight.astype(jnp.float32).reshape(NW, 1, E_PER_W)
    layers = [(W0, b0), (W1, b1), (W2, b2)]
    outs = [h]
    cur = h
    for l, (W, b) in enumerate(layers):
        part = _sc_aggregate(cur, src3, dst3, w3)
        cur = _tc_layer(part, W, b, act=(l < 2))
        outs.append(cur)
    return jnp.concatenate(outs, axis=1)
